# Initial kernel scaffold; baseline (speedup 1.0000x reference)
import jax
import jax.numpy as jnp
from jax.experimental import pallas as pl


def kernel(x, rbf, edge_index, W_rbf, W1, b1, W2, b2):
    raise NotImplementedError("write your pallas kernel here")


# speedup vs baseline: 1.7980x; 1.7980x over previous
import functools

import jax
import jax.numpy as jnp
from jax import lax
from jax.experimental import pallas as pl
from jax.experimental.pallas import tpu as pltpu
from jax.experimental.pallas import tpu_sc as plsc

N = 10000
E = 320000
HID = 128
NUM_RADIAL = 16

NC = 2
NS = 16
L = 16
NW = NC * NS
EPW = E // NW
C = 80
NCHUNK = EPW // C
ROWS_PER_TILE = 624
TAIL_START = NS * ROWS_PER_TILE
TAIL_ROWS = N - TAIL_START


_BE = 4000


def _emb_body(rbf_ref, wt_ref, out_ref):
    out_ref[...] = jnp.dot(rbf_ref[...], wt_ref[...],
                           preferred_element_type=jnp.float32)


def _emb(rbf, w_rbf_t):
    return pl.pallas_call(
        _emb_body,
        grid=(E // _BE,),
        in_specs=[
            pl.BlockSpec((_BE, NUM_RADIAL), lambda i: (i, 0)),
            pl.BlockSpec((NUM_RADIAL, HID), lambda i: (0, 0)),
        ],
        out_specs=pl.BlockSpec((_BE, HID), lambda i: (i, 0)),
        out_shape=jax.ShapeDtypeStruct((E, HID), jnp.float32),
    )(rbf, w_rbf_t)


_mesh = plsc.VectorSubcoreMesh(core_axis_name="c", subcore_axis_name="s")


@functools.partial(
    pl.kernel,
    out_type=jax.ShapeDtypeStruct((NC, N, HID), jnp.float32),
    mesh=_mesh,
    scratch_types=[
        pltpu.VMEM((C,), jnp.int32),
        pltpu.VMEM((C,), jnp.int32),
        pltpu.VMEM((C, HID), jnp.float32),
        pltpu.VMEM((C, HID), jnp.float32),
        pltpu.VMEM_SHARED((N, HID), jnp.float32),
        pltpu.SemaphoreType.DMA,
    ],
)
def _edge_kernel(x_hbm, emb_hbm, row_hbm, col_hbm, out_hbm,
                 row_v, col_v, xg_v, emb_v, acc_sh, sem):
    c = lax.axis_index("c")
    s = lax.axis_index("s")
    wid = c * NS + s

    def _zero_body(i, carry):
        for j in range(HID // L):
            xg_v[i, pl.ds(j * L, L)] = jnp.zeros((L,), jnp.float32)
        return carry
    lax.fori_loop(0, C, _zero_body, 0)

    r0 = s * ROWS_PER_TILE
    full, rem = divmod(ROWS_PER_TILE, C)
    for k in range(full):
        pltpu.sync_copy(xg_v, acc_sh.at[pl.ds(r0 + k * C, C)])
    if rem:
        pltpu.sync_copy(xg_v.at[pl.ds(0, rem)],
                        acc_sh.at[pl.ds(r0 + full * C, rem)])

    @pl.when(s == NS - 1)
    def _zero_tail():
        pltpu.sync_copy(xg_v.at[pl.ds(0, TAIL_ROWS)],
                        acc_sh.at[pl.ds(TAIL_START, TAIL_ROWS)])

    plsc.subcore_barrier()

    base0 = wid * EPW

    def _chunk(k, carry):
        base = base0 + k * C
        pltpu.sync_copy(row_hbm.at[pl.ds(base, C)], row_v)
        pltpu.sync_copy(col_hbm.at[pl.ds(base, C)], col_v)
        pltpu.async_copy(x_hbm.at[row_v], xg_v, sem).wait()
        pltpu.sync_copy(emb_hbm.at[pl.ds(base, C)], emb_v)

        def _mul(i, inner):
            for j in range(HID // L):
                sl = pl.ds(j * L, L)
                xg_v[i, sl] = xg_v[i, sl] * emb_v[i, sl]
            return inner
        lax.fori_loop(0, C, _mul, 0, unroll=2)

        pltpu.sync_copy(xg_v, acc_sh.at[col_v], add=True)
        return carry

    lax.fori_loop(0, NCHUNK, _chunk, 0)
    plsc.subcore_barrier()

    pltpu.sync_copy(acc_sh.at[pl.ds(r0, ROWS_PER_TILE)],
                    out_hbm.at[c, pl.ds(r0, ROWS_PER_TILE)])

    @pl.when(s == NS - 1)
    def _copy_tail():
        pltpu.sync_copy(acc_sh.at[pl.ds(TAIL_START, TAIL_ROWS)],
                        out_hbm.at[c, pl.ds(TAIL_START, TAIL_ROWS)])


_BN = 2000


def _mlp_body(p_ref, w1_ref, b1_ref, w2_ref, b2_ref, o_ref):
    acc = p_ref[0] + p_ref[1]
    h = jnp.dot(acc, w1_ref[...], preferred_element_type=jnp.float32)
    h = h + b1_ref[...]
    h = h * jax.nn.sigmoid(h)
    o = jnp.dot(h, w2_ref[...], preferred_element_type=jnp.float32)
    o_ref[...] = o + b2_ref[...]


def _mlp(partials, w1_t, b1_2d, w2_t, b2_2d):
    return pl.pallas_call(
        _mlp_body,
        grid=(N // _BN,),
        in_specs=[
            pl.BlockSpec((NC, _BN, HID), lambda i: (0, i, 0)),
            pl.BlockSpec((HID, HID), lambda i: (0, 0)),
            pl.BlockSpec((1, HID), lambda i: (0, 0)),
            pl.BlockSpec((HID, HID), lambda i: (0, 0)),
            pl.BlockSpec((1, HID), lambda i: (0, 0)),
        ],
        out_specs=pl.BlockSpec((_BN, HID), lambda i: (i, 0)),
        out_shape=jax.ShapeDtypeStruct((N, HID), jnp.float32),
    )(partials, w1_t, b1_2d, w2_t, b2_2d)


def kernel(x, rbf, edge_index, W_rbf, W1, b1, W2, b2):
    emb = _emb(rbf, W_rbf.T)
    row = edge_index[0]
    col = edge_index[1]
    partials = _edge_kernel(x, emb, row, col)
    return _mlp(partials, W1.T, b1.reshape(1, -1), W2.T, b2.reshape(1, -1))


# speedup vs baseline: 2.1873x; 1.2165x over previous
import functools

import jax
import jax.numpy as jnp
from jax import lax
from jax.experimental import pallas as pl
from jax.experimental.pallas import tpu as pltpu
from jax.experimental.pallas import tpu_sc as plsc

N = 10000
E = 320000
HID = 128
NUM_RADIAL = 16

NC = 2
NS = 16
L = 16
NW = NC * NS
EPW = E // NW
C = 40
NCHUNK = EPW // C
ROWS_PER_TILE = 624
TAIL_START = NS * ROWS_PER_TILE
TAIL_ROWS = N - TAIL_START


_BE = 4000


def _emb_body(rbf_ref, wt_ref, out_ref):
    out_ref[...] = jnp.dot(rbf_ref[...], wt_ref[...],
                           preferred_element_type=jnp.float32)


def _emb(rbf, w_rbf_t):
    return pl.pallas_call(
        _emb_body,
        grid=(E // _BE,),
        in_specs=[
            pl.BlockSpec((_BE, NUM_RADIAL), lambda i: (i, 0)),
            pl.BlockSpec((NUM_RADIAL, HID), lambda i: (0, 0)),
        ],
        out_specs=pl.BlockSpec((_BE, HID), lambda i: (i, 0)),
        out_shape=jax.ShapeDtypeStruct((E, HID), jnp.float32),
    )(rbf, w_rbf_t)


_mesh = plsc.VectorSubcoreMesh(core_axis_name="c", subcore_axis_name="s")

NBUF = 4
NIBUF = 8


@functools.partial(
    pl.kernel,
    out_type=jax.ShapeDtypeStruct((NC, N, HID), jnp.float32),
    mesh=_mesh,
    scratch_types=(
        [pltpu.VMEM((C,), jnp.int32)] * NIBUF +
        [pltpu.VMEM((C,), jnp.int32)] * NIBUF +
        [pltpu.VMEM((C, HID), jnp.float32)] * NBUF +
        [pltpu.VMEM((C, HID), jnp.float32)] * NBUF +
        [pltpu.VMEM_SHARED((N, HID), jnp.float32)] +
        [pltpu.SemaphoreType.DMA] * (NIBUF + 3 * NBUF)
    ),
)
def _edge_kernel(x_hbm, emb_hbm, row_hbm, col_hbm, out_hbm, *scr):
    row_vs = scr[0:NIBUF]
    col_vs = scr[NIBUF:2 * NIBUF]
    xg_vs = scr[2 * NIBUF:2 * NIBUF + NBUF]
    emb_vs = scr[2 * NIBUF + NBUF:2 * NIBUF + 2 * NBUF]
    acc_sh = scr[2 * NIBUF + 2 * NBUF]
    sems = scr[2 * NIBUF + 2 * NBUF + 1:]
    semI = sems[0:NIBUF]
    semG = sems[NIBUF:NIBUF + NBUF]
    semE = sems[NIBUF + NBUF:NIBUF + 2 * NBUF]
    semS = sems[NIBUF + 2 * NBUF:NIBUF + 3 * NBUF]

    c = lax.axis_index("c")
    s = lax.axis_index("s")
    wid = c * NS + s
    base0 = wid * EPW

    def _zero_body(i, carry):
        for j in range(HID // L):
            xg_vs[0][i, pl.ds(j * L, L)] = jnp.zeros((L,), jnp.float32)
        return carry
    lax.fori_loop(0, C, _zero_body, 0)

    r0 = s * ROWS_PER_TILE
    full, rem = divmod(ROWS_PER_TILE, C)
    for k in range(full):
        pltpu.sync_copy(xg_vs[0], acc_sh.at[pl.ds(r0 + k * C, C)])
    if rem:
        pltpu.sync_copy(xg_vs[0].at[pl.ds(0, rem)],
                        acc_sh.at[pl.ds(r0 + full * C, rem)])

    @pl.when(s == NS - 1)
    def _zero_tail():
        pltpu.sync_copy(xg_vs[0].at[pl.ds(0, TAIL_ROWS)],
                        acc_sh.at[pl.ds(TAIL_START, TAIL_ROWS)])

    plsc.subcore_barrier()

    def _issue_idx(kv, I):
        base = base0 + kv * C
        pltpu.async_copy(row_hbm.at[pl.ds(base, C)], row_vs[I], semI[I])
        pltpu.async_copy(col_hbm.at[pl.ds(base, C)], col_vs[I], semI[I])

    def _wait_idx(I):
        pltpu.make_async_copy(row_hbm.at[pl.ds(0, C)], row_vs[I], semI[I]).wait()
        pltpu.make_async_copy(col_hbm.at[pl.ds(0, C)], col_vs[I], semI[I]).wait()

    def _issue_data(kv, K, I):
        pltpu.async_copy(x_hbm.at[row_vs[I]], xg_vs[K], semG[K])
        base = base0 + kv * C
        pltpu.async_copy(emb_hbm.at[pl.ds(base, C)], emb_vs[K], semE[K])

    def _wait_data(K, I):
        pltpu.make_async_copy(x_hbm.at[row_vs[I]], xg_vs[K], semG[K]).wait()
        pltpu.make_async_copy(emb_hbm.at[pl.ds(0, C)], emb_vs[K], semE[K]).wait()

    def _issue_scatter(K, I):
        pltpu.async_copy(xg_vs[K], acc_sh.at[col_vs[I]], semS[K], add=True)

    def _wait_scatter(K, I):
        pltpu.make_async_copy(xg_vs[K], acc_sh.at[col_vs[I]], semS[K]).wait()

    def _compute(K):
        xg_v = xg_vs[K]
        emb_v = emb_vs[K]

        def _mul(i, inner):
            for j in range(HID // L):
                sl = pl.ds(j * L, L)
                xg_v[i, sl] = xg_v[i, sl] * emb_v[i, sl]
            return inner
        lax.fori_loop(0, C, _mul, 0, unroll=2)

    def _stage(kv, k_static):
        K = k_static % NBUF
        I = k_static % NIBUF
        _wait_data(K, I)
        _compute(K)
        _issue_scatter(K, I)
        if k_static + 1 < NCHUNK:
            Kn = (k_static + 1) % NBUF
            In = (k_static + 1) % NIBUF
            _wait_idx(In)
            if k_static + 1 >= NBUF:
                _wait_scatter(Kn, In)
            _issue_data(kv + 1, Kn, In)
        if k_static + 2 < NCHUNK:
            _issue_idx(kv + 2, (k_static + 2) % NIBUF)

    _issue_idx(0, 0)
    _issue_idx(1, 1)
    _wait_idx(0)
    _issue_data(0, 0, 0)

    for k in range(3):
        _stage(k, k)

    n_main = (NCHUNK - 3 - 2) // NIBUF * NIBUF

    def _main(i, carry):
        for j in range(NIBUF):
            _stage(3 + i * NIBUF + j, 3 + j)
        return carry
    lax.fori_loop(0, n_main // NIBUF, _main, 0)

    for k in range(3 + n_main, NCHUNK):
        _stage(k, k)

    for k in range(NCHUNK - NBUF, NCHUNK):
        _wait_scatter(k % NBUF, k % NIBUF)

    plsc.subcore_barrier()

    pltpu.sync_copy(acc_sh.at[pl.ds(r0, ROWS_PER_TILE)],
                    out_hbm.at[c, pl.ds(r0, ROWS_PER_TILE)])

    @pl.when(s == NS - 1)
    def _copy_tail():
        pltpu.sync_copy(acc_sh.at[pl.ds(TAIL_START, TAIL_ROWS)],
                        out_hbm.at[c, pl.ds(TAIL_START, TAIL_ROWS)])


_BN = 2000


def _mlp_body(p_ref, w1_ref, b1_ref, w2_ref, b2_ref, o_ref):
    acc = p_ref[0] + p_ref[1]
    h = jnp.dot(acc, w1_ref[...], preferred_element_type=jnp.float32)
    h = h + b1_ref[...]
    h = h * jax.nn.sigmoid(h)
    o = jnp.dot(h, w2_ref[...], preferred_element_type=jnp.float32)
    o_ref[...] = o + b2_ref[...]


def _mlp(partials, w1_t, b1_2d, w2_t, b2_2d):
    return pl.pallas_call(
        _mlp_body,
        grid=(N // _BN,),
        in_specs=[
            pl.BlockSpec((NC, _BN, HID), lambda i: (0, i, 0)),
            pl.BlockSpec((HID, HID), lambda i: (0, 0)),
            pl.BlockSpec((1, HID), lambda i: (0, 0)),
            pl.BlockSpec((HID, HID), lambda i: (0, 0)),
            pl.BlockSpec((1, HID), lambda i: (0, 0)),
        ],
        out_specs=pl.BlockSpec((_BN, HID), lambda i: (i, 0)),
        out_shape=jax.ShapeDtypeStruct((N, HID), jnp.float32),
    )(partials, w1_t, b1_2d, w2_t, b2_2d)


def kernel(x, rbf, edge_index, W_rbf, W1, b1, W2, b2):
    emb = _emb(rbf, W_rbf.T)
    row = edge_index[0]
    col = edge_index[1]
    partials = _edge_kernel(x, emb, row, col)
    return _mlp(partials, W1.T, b1.reshape(1, -1), W2.T, b2.reshape(1, -1))


# speedup vs baseline: 2.9839x; 1.3642x over previous
import functools

import jax
import jax.numpy as jnp
from jax import lax
from jax.experimental import pallas as pl
from jax.experimental.pallas import tpu as pltpu
from jax.experimental.pallas import tpu_sc as plsc

N = 10000
E = 320000
HID = 128
NUM_RADIAL = 16

NC = 2
NS = 16
L = 16
NW = NC * NS
EPW = E // NW
C = 40
NCHUNK = EPW // C
ROWS_PER_TILE = 624
TAIL_START = NS * ROWS_PER_TILE
TAIL_ROWS = N - TAIL_START


_BE = 4000


def _emb_body(rbf_ref, wt_ref, out_ref):
    out_ref[...] = jnp.dot(rbf_ref[...], wt_ref[...],
                           preferred_element_type=jnp.float32)


def _emb(rbf, w_rbf_t):
    return pl.pallas_call(
        _emb_body,
        grid=(E // _BE,),
        in_specs=[
            pl.BlockSpec((_BE, NUM_RADIAL), lambda i: (i, 0)),
            pl.BlockSpec((NUM_RADIAL, HID), lambda i: (0, 0)),
        ],
        out_specs=pl.BlockSpec((_BE, HID), lambda i: (i, 0)),
        out_shape=jax.ShapeDtypeStruct((E, HID), jnp.float32),
    )(rbf, w_rbf_t)


_mesh = plsc.VectorSubcoreMesh(core_axis_name="c", subcore_axis_name="s")

NBUF = 4
NIBUF = 8


@functools.partial(
    pl.kernel,
    out_type=jax.ShapeDtypeStruct((NC, N, HID), jnp.float32),
    mesh=_mesh,
    scratch_types=(
        [pltpu.VMEM((C,), jnp.int32)] * NIBUF +
        [pltpu.VMEM((C,), jnp.int32)] * NIBUF +
        [pltpu.VMEM((C, HID), jnp.float32)] * NBUF +
        [pltpu.VMEM((C, HID), jnp.float32)] * NBUF +
        [pltpu.VMEM_SHARED((N, HID), jnp.float32)] +
        [pltpu.SemaphoreType.DMA] * (NIBUF + 3 * NBUF)
    ),
)
def _edge_kernel(x_hbm, emb_hbm, row_hbm, col_hbm, out_hbm, *scr):
    row_vs = scr[0:NIBUF]
    col_vs = scr[NIBUF:2 * NIBUF]
    xg_vs = scr[2 * NIBUF:2 * NIBUF + NBUF]
    emb_vs = scr[2 * NIBUF + NBUF:2 * NIBUF + 2 * NBUF]
    acc_sh = scr[2 * NIBUF + 2 * NBUF]
    sems = scr[2 * NIBUF + 2 * NBUF + 1:]
    semI = sems[0:NIBUF]
    semG = sems[NIBUF:NIBUF + NBUF]
    semE = sems[NIBUF + NBUF:NIBUF + 2 * NBUF]
    semS = sems[NIBUF + 2 * NBUF:NIBUF + 3 * NBUF]

    c = lax.axis_index("c")
    s = lax.axis_index("s")
    wid = c * NS + s
    base0 = wid * EPW

    def _zero_body(i, carry):
        for j in range(HID // L):
            xg_vs[0][i, pl.ds(j * L, L)] = jnp.zeros((L,), jnp.float32)
        return carry
    lax.fori_loop(0, C, _zero_body, 0)

    r0 = s * ROWS_PER_TILE
    full, rem = divmod(ROWS_PER_TILE, C)
    for k in range(full):
        pltpu.sync_copy(xg_vs[0], acc_sh.at[pl.ds(r0 + k * C, C)])
    if rem:
        pltpu.sync_copy(xg_vs[0].at[pl.ds(0, rem)],
                        acc_sh.at[pl.ds(r0 + full * C, rem)])

    @pl.when(s == NS - 1)
    def _zero_tail():
        pltpu.sync_copy(xg_vs[0].at[pl.ds(0, TAIL_ROWS)],
                        acc_sh.at[pl.ds(TAIL_START, TAIL_ROWS)])

    plsc.subcore_barrier()

    def _issue_idx(kv, I):
        base = base0 + kv * C
        pltpu.async_copy(row_hbm.at[pl.ds(base, C)], row_vs[I], semI[I])
        pltpu.async_copy(col_hbm.at[pl.ds(base, C)], col_vs[I], semI[I])

    def _wait_idx(I):
        pltpu.make_async_copy(row_hbm.at[pl.ds(0, C)], row_vs[I], semI[I]).wait()
        pltpu.make_async_copy(col_hbm.at[pl.ds(0, C)], col_vs[I], semI[I]).wait()

    def _issue_data(kv, K, I):
        pltpu.async_copy(x_hbm.at[row_vs[I]], xg_vs[K], semG[K])
        base = base0 + kv * C
        pltpu.async_copy(emb_hbm.at[pl.ds(base, C)], emb_vs[K], semE[K])

    def _wait_data(K, I):
        pltpu.make_async_copy(x_hbm.at[row_vs[I]], xg_vs[K], semG[K]).wait()
        pltpu.make_async_copy(emb_hbm.at[pl.ds(0, C)], emb_vs[K], semE[K]).wait()

    def _issue_scatter(K, I):
        pltpu.async_copy(xg_vs[K], acc_sh.at[col_vs[I]], semS[K], add=True)

    def _wait_scatter(K, I):
        pltpu.make_async_copy(xg_vs[K], acc_sh.at[col_vs[I]], semS[K]).wait()

    def _compute(K):
        xg_v = xg_vs[K]
        emb_v = emb_vs[K]

        def _mul(i, inner):
            for j in range(HID // L):
                sl = pl.ds(j * L, L)
                xg_v[i, sl] = xg_v[i, sl] * emb_v[i, sl]
            return inner
        lax.fori_loop(0, C, _mul, 0, unroll=2)

    def _stage(kv, k_static):
        K = k_static % NBUF
        I = k_static % NIBUF
        _wait_data(K, I)
        _compute(K)
        _issue_scatter(K, I)
        if k_static + 2 < NCHUNK:
            Kn = (k_static + 2) % NBUF
            In = (k_static + 2) % NIBUF
            _wait_idx(In)
            if k_static - 2 >= 0:
                _wait_scatter(Kn, In)
            _issue_data(kv + 2, Kn, In)
        if k_static + 4 < NCHUNK:
            _issue_idx(kv + 4, (k_static + 4) % NIBUF)

    for k in range(4):
        _issue_idx(k, k)
    _wait_idx(0)
    _issue_data(0, 0, 0)
    _wait_idx(1)
    _issue_data(1, 1, 1)

    for k in range(4):
        _stage(k, k)

    n_main = (NCHUNK - 4 - 4) // NIBUF * NIBUF

    def _main(i, carry):
        for j in range(NIBUF):
            _stage(4 + i * NIBUF + j, 4 + j)
        return carry
    lax.fori_loop(0, n_main // NIBUF, _main, 0)

    for k in range(4 + n_main, NCHUNK):
        _stage(k, k)

    for k in range(NCHUNK - NBUF, NCHUNK):
        _wait_scatter(k % NBUF, k % NIBUF)

    plsc.subcore_barrier()

    pltpu.sync_copy(acc_sh.at[pl.ds(r0, ROWS_PER_TILE)],
                    out_hbm.at[c, pl.ds(r0, ROWS_PER_TILE)])

    @pl.when(s == NS - 1)
    def _copy_tail():
        pltpu.sync_copy(acc_sh.at[pl.ds(TAIL_START, TAIL_ROWS)],
                        out_hbm.at[c, pl.ds(TAIL_START, TAIL_ROWS)])


_BN = 2000


def _mlp_body(p_ref, w1_ref, b1_ref, w2_ref, b2_ref, o_ref):
    acc = p_ref[0] + p_ref[1]
    h = jnp.dot(acc, w1_ref[...], preferred_element_type=jnp.float32)
    h = h + b1_ref[...]
    h = h * jax.nn.sigmoid(h)
    o = jnp.dot(h, w2_ref[...], preferred_element_type=jnp.float32)
    o_ref[...] = o + b2_ref[...]


def _mlp(partials, w1_t, b1_2d, w2_t, b2_2d):
    return pl.pallas_call(
        _mlp_body,
        grid=(N // _BN,),
        in_specs=[
            pl.BlockSpec((NC, _BN, HID), lambda i: (0, i, 0)),
            pl.BlockSpec((HID, HID), lambda i: (0, 0)),
            pl.BlockSpec((1, HID), lambda i: (0, 0)),
            pl.BlockSpec((HID, HID), lambda i: (0, 0)),
            pl.BlockSpec((1, HID), lambda i: (0, 0)),
        ],
        out_specs=pl.BlockSpec((_BN, HID), lambda i: (i, 0)),
        out_shape=jax.ShapeDtypeStruct((N, HID), jnp.float32),
    )(partials, w1_t, b1_2d, w2_t, b2_2d)


def kernel(x, rbf, edge_index, W_rbf, W1, b1, W2, b2):
    emb = _emb(rbf, W_rbf.T)
    row = edge_index[0]
    col = edge_index[1]
    partials = _edge_kernel(x, emb, row, col)
    return _mlp(partials, W1.T, b1.reshape(1, -1), W2.T, b2.reshape(1, -1))


# speedup vs baseline: 4.2688x; 1.4306x over previous
import functools

import jax
import jax.numpy as jnp
from jax import lax
from jax.experimental import pallas as pl
from jax.experimental.pallas import tpu as pltpu
from jax.experimental.pallas import tpu_sc as plsc

N = 10000
E = 320000
HID = 128
NUM_RADIAL = 16

NC = 2
NS = 16
L = 16
NW = NC * NS
EPW = E // NW
C = 40
NCHUNK = EPW // C
ROWS_PER_TILE = 624
TAIL_START = NS * ROWS_PER_TILE
TAIL_ROWS = N - TAIL_START


_BE = 4000


def _emb_body(rbf_ref, wt_ref, out_ref):
    out_ref[...] = jnp.dot(rbf_ref[...], wt_ref[...],
                           preferred_element_type=jnp.float32)


def _emb(rbf, w_rbf_t):
    return pl.pallas_call(
        _emb_body,
        grid=(E // _BE,),
        in_specs=[
            pl.BlockSpec((_BE, NUM_RADIAL), lambda i: (i, 0)),
            pl.BlockSpec((NUM_RADIAL, HID), lambda i: (0, 0)),
        ],
        out_specs=pl.BlockSpec((_BE, HID), lambda i: (i, 0)),
        out_shape=jax.ShapeDtypeStruct((E, HID), jnp.float32),
    )(rbf, w_rbf_t)


_mesh = plsc.VectorSubcoreMesh(core_axis_name="c", subcore_axis_name="s")

NBUF = 4
NIBUF = 8


@functools.partial(
    pl.kernel,
    out_type=jax.ShapeDtypeStruct((NC, N, HID), jnp.float32),
    mesh=_mesh,
    scratch_types=(
        [pltpu.VMEM((C,), jnp.int32)] * NIBUF +
        [pltpu.VMEM((C,), jnp.int32)] * NIBUF +
        [pltpu.VMEM((C, HID), jnp.float32)] * NBUF +
        [pltpu.VMEM((C, HID), jnp.float32)] * NBUF +
        [pltpu.VMEM_SHARED((N, HID), jnp.float32)] +
        [pltpu.SemaphoreType.DMA] * (NIBUF + 3 * NBUF)
    ),
)
def _edge_kernel(x_hbm, emb_hbm, row_hbm, col_hbm, out_hbm, *scr):
    row_vs = scr[0:NIBUF]
    col_vs = scr[NIBUF:2 * NIBUF]
    xg_vs = scr[2 * NIBUF:2 * NIBUF + NBUF]
    emb_vs = scr[2 * NIBUF + NBUF:2 * NIBUF + 2 * NBUF]
    acc_sh = scr[2 * NIBUF + 2 * NBUF]
    sems = scr[2 * NIBUF + 2 * NBUF + 1:]
    semI = sems[0:NIBUF]
    semG = sems[NIBUF:NIBUF + NBUF]
    semE = sems[NIBUF + NBUF:NIBUF + 2 * NBUF]
    semS = sems[NIBUF + 2 * NBUF:NIBUF + 3 * NBUF]

    c = lax.axis_index("c")
    s = lax.axis_index("s")
    wid = c * NS + s
    base0 = wid * EPW

    def _zero_body(i, carry):
        for j in range(HID // L):
            xg_vs[0][i, pl.ds(j * L, L)] = jnp.zeros((L,), jnp.float32)
        return carry
    lax.fori_loop(0, C, _zero_body, 0)

    r0 = s * ROWS_PER_TILE
    full, rem = divmod(ROWS_PER_TILE, C)
    for k in range(full):
        pltpu.sync_copy(xg_vs[0], acc_sh.at[pl.ds(r0 + k * C, C)])
    if rem:
        pltpu.sync_copy(xg_vs[0].at[pl.ds(0, rem)],
                        acc_sh.at[pl.ds(r0 + full * C, rem)])

    @pl.when(s == NS - 1)
    def _zero_tail():
        pltpu.sync_copy(xg_vs[0].at[pl.ds(0, TAIL_ROWS)],
                        acc_sh.at[pl.ds(TAIL_START, TAIL_ROWS)])

    plsc.subcore_barrier()

    def _issue_idx(kv, I):
        base = base0 + kv * C
        pltpu.async_copy(row_hbm.at[pl.ds(base, C)], row_vs[I], semI[I])
        pltpu.async_copy(col_hbm.at[pl.ds(base, C)], col_vs[I], semI[I])

    def _wait_idx(I):
        pltpu.make_async_copy(row_hbm.at[pl.ds(0, C)], row_vs[I], semI[I]).wait()
        pltpu.make_async_copy(col_hbm.at[pl.ds(0, C)], col_vs[I], semI[I]).wait()

    def _issue_data(kv, K, I):
        pltpu.async_copy(x_hbm.at[row_vs[I]], xg_vs[K], semG[K])
        base = base0 + kv * C
        pltpu.async_copy(emb_hbm.at[pl.ds(base, C)], emb_vs[K], semE[K])

    def _wait_data(K, I):
        pltpu.make_async_copy(x_hbm.at[row_vs[I]], xg_vs[K], semG[K]).wait()
        pltpu.make_async_copy(emb_hbm.at[pl.ds(0, C)], emb_vs[K], semE[K]).wait()

    def _issue_scatter(K, I):
        pltpu.async_copy(xg_vs[K], acc_sh.at[col_vs[I]], semS[K], add=True)

    def _wait_scatter(K, I):
        pltpu.make_async_copy(xg_vs[K], acc_sh.at[col_vs[I]], semS[K]).wait()

    def _compute(K):
        xg_v = xg_vs[K]
        emb_v = emb_vs[K]

        @plsc.parallel_loop(0, C, unroll=4)
        def _mul(i):
            for j in range(HID // L):
                sl = pl.ds(j * L, L)
                xg_v[i, sl] = xg_v[i, sl] * emb_v[i, sl]

    def _stage(kv, k_static):
        K = k_static % NBUF
        I = k_static % NIBUF
        _wait_data(K, I)
        _compute(K)
        _issue_scatter(K, I)
        if k_static + 2 < NCHUNK:
            Kn = (k_static + 2) % NBUF
            In = (k_static + 2) % NIBUF
            _wait_idx(In)
            if k_static - 2 >= 0:
                _wait_scatter(Kn, In)
            _issue_data(kv + 2, Kn, In)
        if k_static + 4 < NCHUNK:
            _issue_idx(kv + 4, (k_static + 4) % NIBUF)

    for k in range(4):
        _issue_idx(k, k)
    _wait_idx(0)
    _issue_data(0, 0, 0)
    _wait_idx(1)
    _issue_data(1, 1, 1)

    for k in range(4):
        _stage(k, k)

    n_main = (NCHUNK - 4 - 4) // NIBUF * NIBUF

    def _main(i, carry):
        for j in range(NIBUF):
            _stage(4 + i * NIBUF + j, 4 + j)
        return carry
    lax.fori_loop(0, n_main // NIBUF, _main, 0)

    for k in range(4 + n_main, NCHUNK):
        _stage(k, k)

    for k in range(NCHUNK - NBUF, NCHUNK):
        _wait_scatter(k % NBUF, k % NIBUF)

    plsc.subcore_barrier()

    pltpu.sync_copy(acc_sh.at[pl.ds(r0, ROWS_PER_TILE)],
                    out_hbm.at[c, pl.ds(r0, ROWS_PER_TILE)])

    @pl.when(s == NS - 1)
    def _copy_tail():
        pltpu.sync_copy(acc_sh.at[pl.ds(TAIL_START, TAIL_ROWS)],
                        out_hbm.at[c, pl.ds(TAIL_START, TAIL_ROWS)])


_BN = 2000


def _mlp_body(p_ref, w1_ref, b1_ref, w2_ref, b2_ref, o_ref):
    acc = p_ref[0] + p_ref[1]
    h = jnp.dot(acc, w1_ref[...], preferred_element_type=jnp.float32)
    h = h + b1_ref[...]
    h = h * jax.nn.sigmoid(h)
    o = jnp.dot(h, w2_ref[...], preferred_element_type=jnp.float32)
    o_ref[...] = o + b2_ref[...]


def _mlp(partials, w1_t, b1_2d, w2_t, b2_2d):
    return pl.pallas_call(
        _mlp_body,
        grid=(N // _BN,),
        in_specs=[
            pl.BlockSpec((NC, _BN, HID), lambda i: (0, i, 0)),
            pl.BlockSpec((HID, HID), lambda i: (0, 0)),
            pl.BlockSpec((1, HID), lambda i: (0, 0)),
            pl.BlockSpec((HID, HID), lambda i: (0, 0)),
            pl.BlockSpec((1, HID), lambda i: (0, 0)),
        ],
        out_specs=pl.BlockSpec((_BN, HID), lambda i: (i, 0)),
        out_shape=jax.ShapeDtypeStruct((N, HID), jnp.float32),
    )(partials, w1_t, b1_2d, w2_t, b2_2d)


def kernel(x, rbf, edge_index, W_rbf, W1, b1, W2, b2):
    emb = _emb(rbf, W_rbf.T)
    row = edge_index[0]
    col = edge_index[1]
    partials = _edge_kernel(x, emb, row, col)
    return _mlp(partials, W1.T, b1.reshape(1, -1), W2.T, b2.reshape(1, -1))


# speedup vs baseline: 4.3522x; 1.0195x over previous
import functools

import jax
import jax.numpy as jnp
from jax import lax
from jax.experimental import pallas as pl
from jax.experimental.pallas import tpu as pltpu
from jax.experimental.pallas import tpu_sc as plsc

N = 10000
E = 320000
HID = 128
NUM_RADIAL = 16

NC = 2
NS = 16
L = 16
NW = NC * NS
EPW = E // NW
C = 40
NCHUNK = EPW // C
ROWS_PER_TILE = 624
TAIL_START = NS * ROWS_PER_TILE
TAIL_ROWS = N - TAIL_START


_BE = 4000


def _emb_body(rbf_ref, wt_ref, out_ref):
    out_ref[...] = jnp.dot(rbf_ref[...], wt_ref[...],
                           preferred_element_type=jnp.float32)


def _emb(rbf, w_rbf_t):
    ne = rbf.shape[0]
    return pl.pallas_call(
        _emb_body,
        grid=(ne // _BE,),
        in_specs=[
            pl.BlockSpec((_BE, NUM_RADIAL), lambda i: (i, 0)),
            pl.BlockSpec((NUM_RADIAL, HID), lambda i: (0, 0)),
        ],
        out_specs=pl.BlockSpec((_BE, HID), lambda i: (i, 0)),
        out_shape=jax.ShapeDtypeStruct((ne, HID), jnp.float32),
    )(rbf, w_rbf_t)


_mesh = plsc.VectorSubcoreMesh(core_axis_name="c", subcore_axis_name="s")

NBUF = 4
NIBUF = 8


def _make_edge_kernel(ne):
  epw = ne // NW
  nchunk = epw // C

  @functools.partial(
      pl.kernel,
      out_type=jax.ShapeDtypeStruct((NC, N, HID), jnp.float32),
      mesh=_mesh,
      scratch_types=(
          [pltpu.VMEM((C,), jnp.int32)] * NIBUF +
          [pltpu.VMEM((C,), jnp.int32)] * NIBUF +
          [pltpu.VMEM((C, HID), jnp.float32)] * NBUF +
          [pltpu.VMEM((C, HID), jnp.float32)] * NBUF +
          [pltpu.VMEM_SHARED((N, HID), jnp.float32)] +
          [pltpu.SemaphoreType.DMA] * (NIBUF + 3 * NBUF)
      ),
  )
  def _edge_kernel(x_hbm, emb_hbm, row_hbm, col_hbm, out_hbm, *scr):
    EPW = epw
    NCHUNK = nchunk
    row_vs = scr[0:NIBUF]
    col_vs = scr[NIBUF:2 * NIBUF]
    xg_vs = scr[2 * NIBUF:2 * NIBUF + NBUF]
    emb_vs = scr[2 * NIBUF + NBUF:2 * NIBUF + 2 * NBUF]
    acc_sh = scr[2 * NIBUF + 2 * NBUF]
    sems = scr[2 * NIBUF + 2 * NBUF + 1:]
    semI = sems[0:NIBUF]
    semG = sems[NIBUF:NIBUF + NBUF]
    semE = sems[NIBUF + NBUF:NIBUF + 2 * NBUF]
    semS = sems[NIBUF + 2 * NBUF:NIBUF + 3 * NBUF]

    c = lax.axis_index("c")
    s = lax.axis_index("s")
    wid = c * NS + s
    base0 = wid * EPW

    def _zero_body(i, carry):
        for j in range(HID // L):
            xg_vs[0][i, pl.ds(j * L, L)] = jnp.zeros((L,), jnp.float32)
        return carry
    lax.fori_loop(0, C, _zero_body, 0)

    r0 = s * ROWS_PER_TILE
    full, rem = divmod(ROWS_PER_TILE, C)
    for k in range(full):
        pltpu.sync_copy(xg_vs[0], acc_sh.at[pl.ds(r0 + k * C, C)])
    if rem:
        pltpu.sync_copy(xg_vs[0].at[pl.ds(0, rem)],
                        acc_sh.at[pl.ds(r0 + full * C, rem)])

    @pl.when(s == NS - 1)
    def _zero_tail():
        pltpu.sync_copy(xg_vs[0].at[pl.ds(0, TAIL_ROWS)],
                        acc_sh.at[pl.ds(TAIL_START, TAIL_ROWS)])

    plsc.subcore_barrier()

    def _issue_idx(kv, I):
        base = base0 + kv * C
        pltpu.async_copy(row_hbm.at[pl.ds(base, C)], row_vs[I], semI[I])
        pltpu.async_copy(col_hbm.at[pl.ds(base, C)], col_vs[I], semI[I])

    def _wait_idx(I):
        pltpu.make_async_copy(row_hbm.at[pl.ds(0, C)], row_vs[I], semI[I]).wait()
        pltpu.make_async_copy(col_hbm.at[pl.ds(0, C)], col_vs[I], semI[I]).wait()

    def _issue_data(kv, K, I):
        pltpu.async_copy(x_hbm.at[row_vs[I]], xg_vs[K], semG[K])
        base = base0 + kv * C
        pltpu.async_copy(emb_hbm.at[pl.ds(base, C)], emb_vs[K], semE[K])

    def _wait_data(K, I):
        pltpu.make_async_copy(x_hbm.at[row_vs[I]], xg_vs[K], semG[K]).wait()
        pltpu.make_async_copy(emb_hbm.at[pl.ds(0, C)], emb_vs[K], semE[K]).wait()

    def _issue_scatter(K, I):
        pltpu.async_copy(xg_vs[K], acc_sh.at[col_vs[I]], semS[K], add=True)

    def _wait_scatter(K, I):
        pltpu.make_async_copy(xg_vs[K], acc_sh.at[col_vs[I]], semS[K]).wait()

    def _compute(K):
        xg_v = xg_vs[K]
        emb_v = emb_vs[K]

        @plsc.parallel_loop(0, C, unroll=4)
        def _mul(i):
            for j in range(HID // L):
                sl = pl.ds(j * L, L)
                xg_v[i, sl] = xg_v[i, sl] * emb_v[i, sl]

    def _stage(kv, k_static):
        K = k_static % NBUF
        I = k_static % NIBUF
        _wait_data(K, I)
        _compute(K)
        _issue_scatter(K, I)
        if k_static + 2 < NCHUNK:
            Kn = (k_static + 2) % NBUF
            In = (k_static + 2) % NIBUF
            _wait_idx(In)
            if k_static - 2 >= 0:
                _wait_scatter(Kn, In)
            _issue_data(kv + 2, Kn, In)
        if k_static + 4 < NCHUNK:
            _issue_idx(kv + 4, (k_static + 4) % NIBUF)

    for k in range(4):
        _issue_idx(k, k)
    _wait_idx(0)
    _issue_data(0, 0, 0)
    _wait_idx(1)
    _issue_data(1, 1, 1)

    for k in range(4):
        _stage(k, k)

    n_main = (NCHUNK - 4 - 4) // NIBUF * NIBUF

    def _main(i, carry):
        for j in range(NIBUF):
            _stage(4 + i * NIBUF + j, 4 + j)
        return carry
    lax.fori_loop(0, n_main // NIBUF, _main, 0)

    for k in range(4 + n_main, NCHUNK):
        _stage(k, k)

    for k in range(NCHUNK - NBUF, NCHUNK):
        _wait_scatter(k % NBUF, k % NIBUF)

    plsc.subcore_barrier()

    pltpu.sync_copy(acc_sh.at[pl.ds(r0, ROWS_PER_TILE)],
                    out_hbm.at[c, pl.ds(r0, ROWS_PER_TILE)])

    @pl.when(s == NS - 1)
    def _copy_tail():
        pltpu.sync_copy(acc_sh.at[pl.ds(TAIL_START, TAIL_ROWS)],
                        out_hbm.at[c, pl.ds(TAIL_START, TAIL_ROWS)])

  return _edge_kernel


NSPLIT = 2
_edge_split = _make_edge_kernel(E // NSPLIT)


_BN = 2000


def _mlp_body(p0_ref, p1_ref, w1_ref, b1_ref, w2_ref, b2_ref, o_ref):
    acc = (p0_ref[0] + p0_ref[1]) + (p1_ref[0] + p1_ref[1])
    h = jnp.dot(acc, w1_ref[...], preferred_element_type=jnp.float32)
    h = h + b1_ref[...]
    h = h * jax.nn.sigmoid(h)
    o = jnp.dot(h, w2_ref[...], preferred_element_type=jnp.float32)
    o_ref[...] = o + b2_ref[...]


def _mlp(p0, p1, w1_t, b1_2d, w2_t, b2_2d):
    return pl.pallas_call(
        _mlp_body,
        grid=(N // _BN,),
        in_specs=[
            pl.BlockSpec((NC, _BN, HID), lambda i: (0, i, 0)),
            pl.BlockSpec((NC, _BN, HID), lambda i: (0, i, 0)),
            pl.BlockSpec((HID, HID), lambda i: (0, 0)),
            pl.BlockSpec((1, HID), lambda i: (0, 0)),
            pl.BlockSpec((HID, HID), lambda i: (0, 0)),
            pl.BlockSpec((1, HID), lambda i: (0, 0)),
        ],
        out_specs=pl.BlockSpec((_BN, HID), lambda i: (i, 0)),
        out_shape=jax.ShapeDtypeStruct((N, HID), jnp.float32),
    )(p0, p1, w1_t, b1_2d, w2_t, b2_2d)


def kernel(x, rbf, edge_index, W_rbf, W1, b1, W2, b2):
    eh = E // NSPLIT
    w_rbf_t = W_rbf.T
    row = edge_index[0]
    col = edge_index[1]
    emb0 = _emb(rbf[:eh], w_rbf_t)
    emb1 = _emb(rbf[eh:], w_rbf_t)
    p0 = _edge_split(x, emb0, row[:eh], col[:eh])
    p1 = _edge_split(x, emb1, row[eh:], col[eh:])
    return _mlp(p0, p1, W1.T, b1.reshape(1, -1), W2.T, b2.reshape(1, -1))
